# baseline (device time: 163675 ns/iter reference)
import jax
import jax.numpy as jnp
from jax import lax
from jax.experimental import pallas as pl
from jax.experimental.pallas import tpu as pltpu

N_DEV = 8
N_EXP = 32
E_LOC = 4
CAP = 96
ROWS = E_LOC * CAP
T = 2048
D = 1024
F = 2048
RB = 2
NSEM = N_DEV * E_LOC


def _sem(k, e):
    return k * E_LOC + e


def _stage_weights(e, w1_ref, w2_ref, w1f, w2f, w1bs, w2bs, w_sems):
    for half in range(2):
        c1 = pltpu.make_async_copy(
            w1_ref.at[e, pl.ds(half * (D // 2), D // 2), :], w1f, w_sems.at[0])
        c2 = pltpu.make_async_copy(
            w2_ref.at[e, pl.ds(half * (F // 2), F // 2), :], w2f, w_sems.at[1])
        c1.start()
        c2.start()
        c1.wait()
        c2.wait()
        w1bs[half * (D // 2):(half + 1) * (D // 2), :] = (
            w1f[...].astype(jnp.bfloat16))
        w2bs[half * (F // 2):(half + 1) * (F // 2), :] = (
            w2f[...].astype(jnp.bfloat16))


def _moe_body(x_ref, g_ref, w1_ref, w2_ref, out_ref,
              send_ref, recv_ref, res_ref,
              w1f, w2f, w1bs, w2bs,
              d_send, d_recv, r_send, r_recv, loc_sems, w_sems):
    me = lax.axis_index("i")

    xv = x_ref[...]
    iota_t = lax.broadcasted_iota(jnp.int32, (ROWS, T), 1)

    d_rdmas = []
    for k in range(1, N_DEV):
        dst = lax.rem(me + k, N_DEV)
        gj = g_ref[pl.ds(dst * ROWS, ROWS), :]
        p = (gj == iota_t).astype(jnp.bfloat16)
        chunk = jnp.dot(p, xv, preferred_element_type=jnp.float32)
        send_ref[pl.ds(dst, 1)] = chunk.astype(jnp.bfloat16)[None]
        for e in range(E_LOC):
            sl = pl.ds(e * CAP, CAP)
            rdma = pltpu.make_async_remote_copy(
                src_ref=send_ref.at[dst, sl, :],
                dst_ref=recv_ref.at[me, sl, :],
                send_sem=d_send.at[_sem(k, e)],
                recv_sem=d_recv.at[_sem(k, e)],
                device_id=(dst,),
                device_id_type=pl.DeviceIdType.MESH,
            )
            rdma.start()
            d_rdmas.append(rdma)

    gme = g_ref[pl.ds(me * ROWS, ROWS), :]
    pme = (gme == iota_t).astype(jnp.bfloat16)
    chunk = jnp.dot(pme, xv, preferred_element_type=jnp.float32)
    recv_ref[pl.ds(me, 1)] = chunk.astype(jnp.bfloat16)[None]

    _stage_weights(0, w1_ref, w2_ref, w1f, w2f, w1bs, w2bs, w_sems)

    r_rdmas = []
    for e in range(E_LOC):
        sl = pl.ds(e * CAP, CAP)
        sl_s = slice(e * CAP, (e + 1) * CAP)
        for k in range(1, N_DEV):
            src = lax.rem(me - k + N_DEV, N_DEV)
            recv = pltpu.make_async_remote_copy(
                src_ref=send_ref.at[me, sl, :],
                dst_ref=recv_ref.at[src, sl, :],
                send_sem=loc_sems.at[0],
                recv_sem=d_recv.at[_sem(k, e)],
                device_id=(src,),
                device_id_type=pl.DeviceIdType.MESH,
            )
            recv.wait_recv()

        n_src = N_DEV // RB
        for b in range(RB):
            srcs = range(b * n_src, (b + 1) * n_src)
            a = jnp.concatenate([recv_ref[s, sl_s, :] for s in srcs], axis=0)
            h = jnp.maximum(
                jnp.dot(a, w1bs[...], preferred_element_type=jnp.float32), 0.0
            ).astype(jnp.bfloat16)
            o = jnp.dot(
                h, w2bs[...], preferred_element_type=jnp.float32
            ).astype(jnp.bfloat16)
            for i, s in enumerate(srcs):
                res_ref[s, sl_s, :] = o[i * CAP:(i + 1) * CAP, :]

        if e + 1 < E_LOC:
            _stage_weights(e + 1, w1_ref, w2_ref, w1f, w2f, w1bs, w2bs, w_sems)

        for k in range(1, N_DEV):
            dst = lax.rem(me + k, N_DEV)
            rdma = pltpu.make_async_remote_copy(
                src_ref=res_ref.at[dst, sl, :],
                dst_ref=out_ref.at[me, sl, :],
                send_sem=r_send.at[_sem(k, e)],
                recv_sem=r_recv.at[_sem(k, e)],
                device_id=(dst,),
                device_id_type=pl.DeviceIdType.MESH,
            )
            rdma.start()
            r_rdmas.append(rdma)

    loc2 = pltpu.make_async_copy(res_ref.at[me], out_ref.at[me], loc_sems.at[0])
    loc2.start()
    loc2.wait()

    for e in range(E_LOC):
        sl = pl.ds(e * CAP, CAP)
        for k in range(1, N_DEV):
            src = lax.rem(me - k + N_DEV, N_DEV)
            recv = pltpu.make_async_remote_copy(
                src_ref=res_ref.at[me, sl, :],
                dst_ref=out_ref.at[src, sl, :],
                send_sem=loc_sems.at[0],
                recv_sem=r_recv.at[_sem(k, e)],
                device_id=(src,),
                device_id_type=pl.DeviceIdType.MESH,
            )
            recv.wait_recv()
    for r in d_rdmas:
        r.wait_send()
    for r in r_rdmas:
        r.wait_send()


def kernel(x, assign, W1, W2):
    assign = assign.astype(jnp.int32)
    onehot = assign[:, None] == jnp.arange(N_EXP, dtype=jnp.int32)[None, :]
    ranks = jnp.cumsum(onehot.astype(jnp.int32), axis=0) - 1
    rank = jnp.sum(jnp.where(onehot, ranks, 0), axis=1)
    slot = assign * CAP + rank

    g = jnp.zeros((N_DEV * ROWS,), jnp.int32)
    g = g.at[slot].set(jnp.arange(T, dtype=jnp.int32), mode="drop",
                       unique_indices=True)

    ret = pl.pallas_call(
        _moe_body,
        out_shape=jax.ShapeDtypeStruct((N_DEV, ROWS, D), jnp.bfloat16),
        in_specs=[
            pl.BlockSpec(memory_space=pltpu.VMEM),
            pl.BlockSpec(memory_space=pltpu.VMEM),
            pl.BlockSpec(memory_space=pl.ANY),
            pl.BlockSpec(memory_space=pl.ANY),
        ],
        out_specs=pl.BlockSpec(memory_space=pltpu.VMEM),
        scratch_shapes=[
            pltpu.VMEM((N_DEV, ROWS, D), jnp.bfloat16),
            pltpu.VMEM((N_DEV, ROWS, D), jnp.bfloat16),
            pltpu.VMEM((N_DEV, ROWS, D), jnp.bfloat16),
            pltpu.VMEM((D // 2, F), jnp.float32),
            pltpu.VMEM((F // 2, D), jnp.float32),
            pltpu.VMEM((D, F), jnp.bfloat16),
            pltpu.VMEM((F, D), jnp.bfloat16),
            pltpu.SemaphoreType.DMA((NSEM,)),
            pltpu.SemaphoreType.DMA((NSEM,)),
            pltpu.SemaphoreType.DMA((NSEM,)),
            pltpu.SemaphoreType.DMA((NSEM,)),
            pltpu.SemaphoreType.DMA((1,)),
            pltpu.SemaphoreType.DMA((2,)),
        ],
        compiler_params=pltpu.CompilerParams(vmem_limit_bytes=62 * 2**20),
    )(x.astype(jnp.bfloat16), g[:, None], W1, W2)

    return ret.reshape(N_DEV * ROWS, D)[slot]


# device time: 148035 ns/iter; 1.1057x vs baseline; 1.1057x over previous
import jax
import jax.numpy as jnp
from jax import lax
from jax.experimental import pallas as pl
from jax.experimental.pallas import tpu as pltpu

N_DEV = 8
N_EXP = 32
E_LOC = 4
CAP = 96
ROWS = E_LOC * CAP
T = 2048
D = 1024
F = 2048
RB = 2
TB = 512
NSEM = N_DEV * E_LOC


def _sem(k, e):
    return k * E_LOC + e


def _stage_weights(e, w1_ref, w2_ref, w1f, w2f, w1bs, w2bs, w_sems):
    for half in range(2):
        c1 = pltpu.make_async_copy(
            w1_ref.at[e, pl.ds(half * (D // 2), D // 2), :], w1f, w_sems.at[0])
        c2 = pltpu.make_async_copy(
            w2_ref.at[e, pl.ds(half * (F // 2), F // 2), :], w2f, w_sems.at[1])
        c1.start()
        c2.start()
        c1.wait()
        c2.wait()
        w1bs[half * (D // 2):(half + 1) * (D // 2), :] = (
            w1f[...].astype(jnp.bfloat16))
        w2bs[half * (F // 2):(half + 1) * (F // 2), :] = (
            w2f[...].astype(jnp.bfloat16))


def _moe_body(x_ref, slot_row_ref, slot_col_ref, w1_ref, w2_ref, out_ref,
              send_ref, recv_ref, res_ref, ret_ref,
              w1f, w2f, w1bs, w2bs,
              d_send, d_recv, r_send, r_recv, loc_sems, w_sems):
    me = lax.axis_index("i")

    xv = x_ref[...]
    slot_row = slot_row_ref[...]
    row_iota = lax.broadcasted_iota(jnp.int32, (ROWS, T), 0)

    d_rdmas = []
    for k in range(1, N_DEV):
        dst = lax.rem(me + k, N_DEV)
        p = (slot_row == row_iota + dst * ROWS).astype(jnp.bfloat16)
        chunk = jnp.dot(p, xv, preferred_element_type=jnp.float32)
        send_ref[k, :, :] = chunk.astype(jnp.bfloat16)
        for e in range(E_LOC):
            sl = pl.ds(e * CAP, CAP)
            rdma = pltpu.make_async_remote_copy(
                src_ref=send_ref.at[k, sl, :],
                dst_ref=recv_ref.at[me, sl, :],
                send_sem=d_send.at[_sem(k, e)],
                recv_sem=d_recv.at[_sem(k, e)],
                device_id=(dst,),
                device_id_type=pl.DeviceIdType.MESH,
            )
            rdma.start()
            d_rdmas.append(rdma)

    pme = (slot_row == row_iota + me * ROWS).astype(jnp.bfloat16)
    chunk = jnp.dot(pme, xv, preferred_element_type=jnp.float32)
    send_ref[0, :, :] = chunk.astype(jnp.bfloat16)
    loc = pltpu.make_async_copy(send_ref.at[0], recv_ref.at[me], loc_sems.at[0])
    loc.start()
    loc.wait()

    _stage_weights(0, w1_ref, w2_ref, w1f, w2f, w1bs, w2bs, w_sems)

    r_rdmas = []
    for e in range(E_LOC):
        sl = pl.ds(e * CAP, CAP)
        sl_s = slice(e * CAP, (e + 1) * CAP)
        for k in range(1, N_DEV):
            src = lax.rem(me - k + N_DEV, N_DEV)
            recv = pltpu.make_async_remote_copy(
                src_ref=send_ref.at[0, sl, :],
                dst_ref=recv_ref.at[src, sl, :],
                send_sem=loc_sems.at[0],
                recv_sem=d_recv.at[_sem(k, e)],
                device_id=(src,),
                device_id_type=pl.DeviceIdType.MESH,
            )
            recv.wait_recv()

        n_src = N_DEV // RB
        for b in range(RB):
            srcs = range(b * n_src, (b + 1) * n_src)
            a = jnp.concatenate([recv_ref[s, sl_s, :] for s in srcs], axis=0)
            h = jnp.maximum(
                jnp.dot(a, w1bs[...], preferred_element_type=jnp.float32), 0.0
            ).astype(jnp.bfloat16)
            o = jnp.dot(
                h, w2bs[...], preferred_element_type=jnp.float32
            ).astype(jnp.bfloat16)
            for i, s in enumerate(srcs):
                res_ref[s, sl_s, :] = o[i * CAP:(i + 1) * CAP, :]

        if e + 1 < E_LOC:
            _stage_weights(e + 1, w1_ref, w2_ref, w1f, w2f, w1bs, w2bs, w_sems)

        for k in range(1, N_DEV):
            dst = lax.rem(me + k, N_DEV)
            rdma = pltpu.make_async_remote_copy(
                src_ref=res_ref.at[dst, sl, :],
                dst_ref=ret_ref.at[me, sl, :],
                send_sem=r_send.at[_sem(k, e)],
                recv_sem=r_recv.at[_sem(k, e)],
                device_id=(dst,),
                device_id_type=pl.DeviceIdType.MESH,
            )
            rdma.start()
            r_rdmas.append(rdma)

    col_iota = lax.broadcasted_iota(jnp.int32, (TB, ROWS), 1)

    for tb in range(T // TB):
        sc = slot_col_ref[tb * TB:(tb + 1) * TB, :]
        q = (sc == col_iota + me * ROWS).astype(jnp.bfloat16)
        v = res_ref[pl.ds(me, 1)].reshape(ROWS, D)
        d = jnp.dot(q, v, preferred_element_type=jnp.float32)
        out_ref[tb * TB:(tb + 1) * TB, :] = d.astype(jnp.bfloat16)

    for k in range(1, N_DEV):
        src = lax.rem(me - k + N_DEV, N_DEV)
        for e in range(E_LOC):
            sl = pl.ds(e * CAP, CAP)
            recv = pltpu.make_async_remote_copy(
                src_ref=res_ref.at[me, sl, :],
                dst_ref=ret_ref.at[src, sl, :],
                send_sem=loc_sems.at[0],
                recv_sem=r_recv.at[_sem(k, e)],
                device_id=(src,),
                device_id_type=pl.DeviceIdType.MESH,
            )
            recv.wait_recv()
        for tb in range(T // TB):
            sc = slot_col_ref[tb * TB:(tb + 1) * TB, :]
            q = (sc == col_iota + src * ROWS).astype(jnp.bfloat16)
            v = ret_ref[pl.ds(src, 1)].reshape(ROWS, D)
            d = jnp.dot(q, v, preferred_element_type=jnp.float32)
            out_ref[tb * TB:(tb + 1) * TB, :] = (
                out_ref[tb * TB:(tb + 1) * TB, :] + d.astype(jnp.bfloat16))

    for r in d_rdmas:
        r.wait_send()
    for r in r_rdmas:
        r.wait_send()


def kernel(x, assign, W1, W2):
    assign = assign.astype(jnp.int32)
    onehot = assign[:, None] == jnp.arange(N_EXP, dtype=jnp.int32)[None, :]
    ranks = lax.associative_scan(jnp.add, onehot.astype(jnp.int32), axis=0) - 1
    rank = jnp.sum(jnp.where(onehot, ranks, 0), axis=1)
    slot = assign * CAP + rank

    out = pl.pallas_call(
        _moe_body,
        out_shape=jax.ShapeDtypeStruct((T, D), jnp.bfloat16),
        in_specs=[
            pl.BlockSpec(memory_space=pltpu.VMEM),
            pl.BlockSpec(memory_space=pltpu.VMEM),
            pl.BlockSpec(memory_space=pltpu.VMEM),
            pl.BlockSpec(memory_space=pl.ANY),
            pl.BlockSpec(memory_space=pl.ANY),
        ],
        out_specs=pl.BlockSpec(memory_space=pltpu.VMEM),
        scratch_shapes=[
            pltpu.VMEM((N_DEV, ROWS, D), jnp.bfloat16),
            pltpu.VMEM((N_DEV, ROWS, D), jnp.bfloat16),
            pltpu.VMEM((N_DEV, ROWS, D), jnp.bfloat16),
            pltpu.VMEM((N_DEV, ROWS, D), jnp.bfloat16),
            pltpu.VMEM((D // 2, F), jnp.float32),
            pltpu.VMEM((F // 2, D), jnp.float32),
            pltpu.VMEM((D, F), jnp.bfloat16),
            pltpu.VMEM((F, D), jnp.bfloat16),
            pltpu.SemaphoreType.DMA((NSEM,)),
            pltpu.SemaphoreType.DMA((NSEM,)),
            pltpu.SemaphoreType.DMA((NSEM,)),
            pltpu.SemaphoreType.DMA((NSEM,)),
            pltpu.SemaphoreType.DMA((1,)),
            pltpu.SemaphoreType.DMA((2,)),
        ],
        compiler_params=pltpu.CompilerParams(vmem_limit_bytes=62 * 2**20),
    )(x.astype(jnp.bfloat16), slot[None, :], slot[:, None], W1, W2)

    return out
